# baseline (speedup 1.0000x reference)
import functools

import jax
import jax.numpy as jnp
from jax import lax
from jax.experimental import pallas as pl
from jax.experimental.pallas import tpu as pltpu
from jax.experimental.pallas import tpu_sc as plsc

_N = 10000
_E = 320000
_D = 128
_NC = 2
_NS = 16
_NW = _NC * _NS
_EPT = _E // _NW
_C = 80
_NCHUNK = _EPT // _C
_C2 = 128
_NCHUNK2 = 81
_EPT2 = _C2 * _NCHUNK2
_PE = _NW * _EPT2
_NP = 10240
_RPT = _NP // _NS

_f32 = jnp.float32
_mesh = plsc.VectorSubcoreMesh(core_axis_name="c", subcore_axis_name="s")


_CE = 2000


@functools.partial(
    pl.kernel,
    out_type=jax.ShapeDtypeStruct((2 * _NW * _NP,), _f32),
    mesh=_mesh,
    scratch_types=[
        pltpu.VMEM((2 * _NP,), _f32),
        pltpu.VMEM((_CE,), jnp.int32),
        pltpu.VMEM((_CE,), jnp.int32),
    ],
    compiler_params=pltpu.CompilerParams(needs_layout_passes=False),
)
def _deg_sc(edge_hbm, out_hbm, deg_v, sidx, didx):
    c = lax.axis_index("c")
    s = lax.axis_index("s")
    tid = c * _NS + s
    zero16 = jnp.zeros((16,), _f32)
    one16 = jnp.full((16,), 1.0, _f32)
    offN = jnp.full((16,), _NP, jnp.int32)

    def zb(i, carry):
        deg_v[pl.ds(i * 16, 16)] = zero16
        return carry

    lax.fori_loop(0, 2 * _NP // 16, zb, 0)

    def step(k, carry):
        base = tid * _EPT + k * _CE
        pltpu.sync_copy(edge_hbm.at[pl.ds(base, _CE)], sidx)
        pltpu.sync_copy(edge_hbm.at[pl.ds(_E + base, _CE)], didx)

        def inner(j, c2):
            plsc.addupdate_scatter(deg_v, [sidx[pl.ds(j * 16, 16)]], one16)
            plsc.addupdate_scatter(deg_v, [didx[pl.ds(j * 16, 16)] + offN], one16)
            return c2

        lax.fori_loop(0, _CE // 16, inner, 0)
        return carry

    lax.fori_loop(0, _EPT // _CE, step, 0)
    pltpu.sync_copy(deg_v.at[pl.ds(0, _NP)], out_hbm.at[pl.ds(tid * _NP, _NP)])
    pltpu.sync_copy(deg_v.at[pl.ds(_NP, _NP)], out_hbm.at[pl.ds((_NW + tid) * _NP, _NP)])


@functools.partial(
    pl.kernel,
    out_type=jax.ShapeDtypeStruct((_NC * _NP, _D), _f32),
    mesh=_mesh,
    scratch_types=[
        pltpu.VMEM_SHARED((_NP, _D), _f32),
        pltpu.VMEM((_NCHUNK2, _C2), jnp.int32),
        pltpu.VMEM((_C2,), jnp.int32),
        pltpu.VMEM((_C2,), jnp.int32),
        pltpu.VMEM((_C2, _D), _f32),
        pltpu.VMEM((_C2, _D), _f32),
        pltpu.SemaphoreType.DMA,
        pltpu.SemaphoreType.DMA,
        pltpu.SemaphoreType.DMA,
        pltpu.SemaphoreType.DMA,
        pltpu.SemaphoreType.DMA,
        pltpu.SemaphoreType.DMA,
        pltpu.SemaphoreType.DMA,
    ],
)
def _edge_sc(hs_hbm, edge_hbm, zeros_hbm, out_hbm, acc, didx, sb0, sb1,
             rows0, rows1, isem, xsem0, xsem1, gsem0, gsem1, ssem0, ssem1):
    c = lax.axis_index("c")
    s = lax.axis_index("s")
    tid = c * _NS + s
    ebase = tid * _EPT2

    def fire_didx(k, carry):
        pltpu.async_copy(edge_hbm.at[pl.ds(_PE + ebase + k * _C2, _C2)], didx.at[k], isem)
        return carry

    lax.fori_loop(0, _NCHUNK2, fire_didx, 0)
    pltpu.sync_copy(zeros_hbm.at[pl.ds(s * _RPT, _RPT)], acc.at[pl.ds(s * _RPT, _RPT)])

    def drain_didx(k, carry):
        pltpu.make_async_copy(edge_hbm.at[pl.ds(_PE + ebase + k * _C2, _C2)],
                              didx.at[k], isem).wait()
        return carry

    lax.fori_loop(0, _NCHUNK2, drain_didx, 0)
    plsc.subcore_barrier()

    def load_sidx(k, buf, sem):
        return pltpu.async_copy(edge_hbm.at[pl.ds(ebase + k * _C2, _C2)], buf, sem)

    def drain_sidx(buf, sem):
        pltpu.make_async_copy(edge_hbm.at[pl.ds(0, _C2)], buf, sem).wait()

    def gather(sbuf, buf, sem):
        return pltpu.async_copy(hs_hbm.at[sbuf], buf, sem)

    def scatter(k, buf, sem):
        return pltpu.async_copy(buf, acc.at[didx.at[k]], sem, add=True)

    def drain(buf, sem):
        pltpu.make_async_copy(hs_hbm.at[pl.ds(0, _C2)], buf, sem).wait()

    load_sidx(0, sb0, xsem0).wait()
    load_sidx(1, sb1, xsem1)
    gather(sb0, rows0, gsem0)

    def body(t, carry):
        k0 = 2 * t
        drain(rows0, gsem0)
        load_sidx(k0 + 2, sb0, xsem0)
        s0 = scatter(k0, rows0, ssem0)

        @pl.when(t > 0)
        def _():
            drain(rows1, ssem1)

        drain_sidx(sb1, xsem1)
        g1 = gather(sb1, rows1, gsem1)
        g1.wait()
        load_sidx(k0 + 3, sb1, xsem1)
        scatter(k0 + 1, rows1, ssem1)
        s0.wait()
        drain_sidx(sb0, xsem0)
        gather(sb0, rows0, gsem0)
        return carry

    lax.fori_loop(0, (_NCHUNK2 - 1) // 2, body, 0)

    kl = _NCHUNK2 - 1
    drain(rows0, gsem0)
    sl = scatter(kl, rows0, ssem0)
    drain(rows1, ssem1)
    drain_sidx(sb1, xsem1)
    sl.wait()
    plsc.subcore_barrier()
    pltpu.sync_copy(acc.at[pl.ds(s * _RPT, _RPT)],
                    out_hbm.at[pl.ds(c * _NP + s * _RPT, _RPT)])


def _embed_body(f_ref, w_ref, b_ref, degp_ref, m_ref, h_ref, hs_ref, a_ref, bc_ref):
    h = jnp.dot(f_ref[...], w_ref[...], preferred_element_type=_f32) + b_ref[...]
    d = lax.dot_general(degp_ref[...], m_ref[...],
                        ((( 0,), (0,)), ((), ())),
                        preferred_element_type=_f32)
    a = lax.rsqrt(jnp.maximum(d[:_N, 0:1], 1.0))
    b = lax.rsqrt(jnp.maximum(d[:_N, 1:2], 1.0))
    h_ref[...] = h
    hs_ref[...] = h * a
    a_ref[...] = a
    bc_ref[...] = b


def _embed_tc(feature, w, b2d, degp, mask):
    return pl.pallas_call(
        _embed_body,
        out_shape=(
            jax.ShapeDtypeStruct((_N, _D), _f32),
            jax.ShapeDtypeStruct((_N, _D), _f32),
            jax.ShapeDtypeStruct((_N, 1), _f32),
            jax.ShapeDtypeStruct((_N, 1), _f32),
        ),
    )(feature, w, b2d, degp, mask)


def _bn_block(aggp, bcol, w, bias, gamma, beta, hprev):
    agg = (aggp[0, :_N] + aggp[1, :_N]) * bcol
    z = jnp.dot(agg, w, preferred_element_type=_f32) + bias
    mu = jnp.mean(z, axis=0, keepdims=True)
    zc = z - mu
    var = jnp.mean(zc * zc, axis=0, keepdims=True)
    zn = zc * lax.rsqrt(var + 1e-5) * gamma + beta
    return hprev + jnp.maximum(zn, 0.0)


def _layer_body(aggp_ref, bc_ref, a_ref, w_ref, bias_ref, g_ref, be_ref, hp_ref,
                h_ref, hs_ref):
    h = _bn_block(aggp_ref[...], bc_ref[...], w_ref[...], bias_ref[...],
                  g_ref[...], be_ref[...], hp_ref[...])
    h_ref[...] = h
    hs_ref[...] = h * a_ref[...]


def _layer_tc(aggp, bcol, acol, w, bias, gamma, beta, hprev):
    return pl.pallas_call(
        _layer_body,
        out_shape=(
            jax.ShapeDtypeStruct((_N, _D), _f32),
            jax.ShapeDtypeStruct((_N, _D), _f32),
        ),
    )(aggp, bcol, acol, w, bias, gamma, beta, hprev)


def _final_body(aggp_ref, bc_ref, w_ref, bias_ref, g_ref, be_ref, hp_ref,
                w1_ref, b1_ref, w2_ref, b2_ref, w3_ref, b3_ref, out_ref):
    h = _bn_block(aggp_ref[...], bc_ref[...], w_ref[...], bias_ref[...],
                  g_ref[...], be_ref[...], hp_ref[...])
    r = jnp.maximum(jnp.dot(h, w1_ref[...], preferred_element_type=_f32) + b1_ref[...], 0.0)
    r = jnp.maximum(jnp.dot(r, w2_ref[...], preferred_element_type=_f32) + b2_ref[...], 0.0)
    out_ref[...] = jnp.dot(r, w3_ref[...], preferred_element_type=_f32) + b3_ref[...]


def _final_tc(aggp, bcol, w, bias, gamma, beta, hprev, mlp):
    args = [aggp, bcol, w, bias, gamma, beta, hprev]
    for lp in mlp:
        args.append(lp['W'])
        args.append(lp['b'].reshape(1, -1))
    return pl.pallas_call(
        _final_body,
        out_shape=jax.ShapeDtypeStruct((_N, 7), _f32),
    )(*args)


def kernel(feature, params, edge_index):
    zeros_nd = jnp.zeros((_NP, _D), _f32)
    mask = jnp.concatenate(
        [jnp.tile(jnp.array([[1.0, 0.0]], _f32), (_NW, 1)),
         jnp.tile(jnp.array([[0.0, 1.0]], _f32), (_NW, 1))], axis=0)

    edge_flat = edge_index.reshape(-1)
    pad_n = _PE - _E
    src_pad = jnp.concatenate([edge_index[0], jnp.zeros((pad_n,), jnp.int32)])
    pad_dst = _N + (jnp.arange(pad_n, dtype=jnp.int32) % (_NP - _N))
    dst_pad = jnp.concatenate([edge_index[1], pad_dst])
    edge_pad = jnp.concatenate([src_pad, dst_pad])
    degp = _deg_sc(edge_flat).reshape(2 * _NW, _NP)

    emb = params['emb']
    h, hs, acol, bcol = _embed_tc(feature, emb['W'], emb['b'].reshape(1, _D), degp, mask)

    layers = params['layers']
    out = None
    for i, lp in enumerate(layers):
        aggp = _edge_sc(hs, edge_pad, zeros_nd).reshape(_NC, _NP, _D)
        w = lp['W']
        bias = lp['b'].reshape(1, -1)
        gamma = lp['gamma'].reshape(1, -1)
        beta = lp['beta'].reshape(1, -1)
        if i < len(layers) - 1:
            h, hs = _layer_tc(aggp, bcol, acol, w, bias, gamma, beta, h)
        else:
            out = _final_tc(aggp, bcol, w, bias, gamma, beta, h, params['mlp'])
    return out


# speedup vs baseline: 1.1657x; 1.1657x over previous
import functools

import jax
import jax.numpy as jnp
from jax import lax
from jax.experimental import pallas as pl
from jax.experimental.pallas import tpu as pltpu
from jax.experimental.pallas import tpu_sc as plsc

_N = 10000
_E = 320000
_D = 128
_NC = 2
_NS = 16
_NW = _NC * _NS
_EPT = _E // _NW
_C = 80
_NCHUNK = _EPT // _C
_C2 = 128
_NCHUNK2 = 81
_EPT2 = _C2 * _NCHUNK2
_PE = _NW * _EPT2
_NP = 10240
_RPT = _NP // _NS

_f32 = jnp.float32
_mesh = plsc.VectorSubcoreMesh(core_axis_name="c", subcore_axis_name="s")


_CE = 2000


@functools.partial(
    pl.kernel,
    out_type=jax.ShapeDtypeStruct((2 * _NW * _NP,), _f32),
    mesh=_mesh,
    scratch_types=[
        pltpu.VMEM((2 * _NP,), _f32),
        pltpu.VMEM((_CE,), jnp.int32),
        pltpu.VMEM((_CE,), jnp.int32),
    ],
    compiler_params=pltpu.CompilerParams(needs_layout_passes=False),
)
def _deg_sc(edge_hbm, out_hbm, deg_v, sidx, didx):
    c = lax.axis_index("c")
    s = lax.axis_index("s")
    tid = c * _NS + s
    zero16 = jnp.zeros((16,), _f32)
    one16 = jnp.full((16,), 1.0, _f32)
    offN = jnp.full((16,), _NP, jnp.int32)

    def zb(i, carry):
        deg_v[pl.ds(i * 16, 16)] = zero16
        return carry

    lax.fori_loop(0, 2 * _NP // 16, zb, 0)

    def step(k, carry):
        base = tid * _EPT + k * _CE
        pltpu.sync_copy(edge_hbm.at[pl.ds(base, _CE)], sidx)
        pltpu.sync_copy(edge_hbm.at[pl.ds(_E + base, _CE)], didx)

        def inner(j, c2):
            plsc.addupdate_scatter(deg_v, [sidx[pl.ds(j * 16, 16)]], one16)
            plsc.addupdate_scatter(deg_v, [didx[pl.ds(j * 16, 16)] + offN], one16)
            return c2

        lax.fori_loop(0, _CE // 16, inner, 0)
        return carry

    lax.fori_loop(0, _EPT // _CE, step, 0)
    pltpu.sync_copy(deg_v.at[pl.ds(0, _NP)], out_hbm.at[pl.ds(tid * _NP, _NP)])
    pltpu.sync_copy(deg_v.at[pl.ds(_NP, _NP)], out_hbm.at[pl.ds((_NW + tid) * _NP, _NP)])


@functools.partial(
    pl.kernel,
    out_type=jax.ShapeDtypeStruct((_NC * _NP, _D), _f32),
    mesh=_mesh,
    scratch_types=[
        pltpu.VMEM_SHARED((_NP, _D), _f32),
        pltpu.VMEM((_NCHUNK2, _C2), jnp.int32),
        pltpu.VMEM((_C2,), jnp.int32),
        pltpu.VMEM((_C2,), jnp.int32),
        pltpu.VMEM((_C2, _D), _f32),
        pltpu.VMEM((_C2, _D), _f32),
        pltpu.SemaphoreType.DMA,
        pltpu.SemaphoreType.DMA,
        pltpu.SemaphoreType.DMA,
        pltpu.SemaphoreType.DMA,
        pltpu.SemaphoreType.DMA,
        pltpu.SemaphoreType.DMA,
        pltpu.SemaphoreType.DMA,
    ],
)
def _edge_sc(hs_hbm, edge_hbm, zeros_hbm, out_hbm, acc, didx, sb0, sb1,
             rows0, rows1, isem, xsem0, xsem1, gsem0, gsem1, ssem0, ssem1):
    c = lax.axis_index("c")
    s = lax.axis_index("s")
    tid = c * _NS + s
    ebase = tid * _EPT2

    def fire_didx(k, carry):
        pltpu.async_copy(edge_hbm.at[pl.ds(_PE + ebase + k * _C2, _C2)], didx.at[k], isem)
        return carry

    lax.fori_loop(0, _NCHUNK2, fire_didx, 0)
    pltpu.sync_copy(zeros_hbm.at[pl.ds(s * _RPT, _RPT)], acc.at[pl.ds(s * _RPT, _RPT)])

    def drain_didx(k, carry):
        pltpu.make_async_copy(edge_hbm.at[pl.ds(_PE + ebase + k * _C2, _C2)],
                              didx.at[k], isem).wait()
        return carry

    lax.fori_loop(0, _NCHUNK2, drain_didx, 0)
    plsc.subcore_barrier()

    def load_sidx(k, buf, sem):
        return pltpu.async_copy(edge_hbm.at[pl.ds(ebase + k * _C2, _C2)], buf, sem)

    def drain_sidx(buf, sem):
        pltpu.make_async_copy(edge_hbm.at[pl.ds(0, _C2)], buf, sem).wait()

    def gather(sbuf, buf, sem):
        return pltpu.async_copy(hs_hbm.at[sbuf], buf, sem)

    def scatter(k, buf, sem):
        return pltpu.async_copy(buf, acc.at[didx.at[k]], sem, add=True)

    def drain(buf, sem):
        pltpu.make_async_copy(hs_hbm.at[pl.ds(0, _C2)], buf, sem).wait()

    load_sidx(0, sb0, xsem0).wait()
    load_sidx(1, sb1, xsem1)
    gather(sb0, rows0, gsem0)

    def body(t, carry):
        k0 = 2 * t
        drain(rows0, gsem0)
        load_sidx(k0 + 2, sb0, xsem0)
        s0 = scatter(k0, rows0, ssem0)

        @pl.when(t > 0)
        def _():
            drain(rows1, ssem1)

        drain_sidx(sb1, xsem1)
        g1 = gather(sb1, rows1, gsem1)
        g1.wait()
        load_sidx(k0 + 3, sb1, xsem1)
        scatter(k0 + 1, rows1, ssem1)
        s0.wait()
        drain_sidx(sb0, xsem0)
        gather(sb0, rows0, gsem0)
        return carry

    lax.fori_loop(0, (_NCHUNK2 - 1) // 2, body, 0)

    kl = _NCHUNK2 - 1
    drain(rows0, gsem0)
    sl = scatter(kl, rows0, ssem0)
    drain(rows1, ssem1)
    drain_sidx(sb1, xsem1)
    sl.wait()
    plsc.subcore_barrier()
    pltpu.sync_copy(acc.at[pl.ds(s * _RPT, _RPT)],
                    out_hbm.at[pl.ds(c * _NP + s * _RPT, _RPT)])


def _embed_body(f_ref, w_ref, b_ref, degp_ref, m_ref, h_ref, hs_ref, a_ref, bc_ref):
    h = jnp.dot(f_ref[...], w_ref[...], preferred_element_type=_f32) + b_ref[...]
    d = lax.dot_general(degp_ref[...], m_ref[...],
                        ((( 0,), (0,)), ((), ())),
                        preferred_element_type=_f32)
    a = lax.rsqrt(jnp.maximum(d[:_N, 0:1], 1.0))
    b = lax.rsqrt(jnp.maximum(d[:_N, 1:2], 1.0))
    h_ref[...] = h
    hs_ref[...] = h * a
    a_ref[...] = a
    bc_ref[...] = b


def _embed_tc(feature, w, b2d, degp, mask):
    return pl.pallas_call(
        _embed_body,
        out_shape=(
            jax.ShapeDtypeStruct((_N, _D), _f32),
            jax.ShapeDtypeStruct((_N, _D), _f32),
            jax.ShapeDtypeStruct((_N, 1), _f32),
            jax.ShapeDtypeStruct((_N, 1), _f32),
        ),
    )(feature, w, b2d, degp, mask)


def _bn_block(aggp, bcol, w, bias, gamma, beta, hprev):
    agg = (aggp[0, :_N] + aggp[1, :_N]) * bcol
    z = jnp.dot(agg, w, preferred_element_type=_f32) + bias
    mu = jnp.mean(z, axis=0, keepdims=True)
    zc = z - mu
    var = jnp.mean(zc * zc, axis=0, keepdims=True)
    zn = zc * lax.rsqrt(var + 1e-5) * gamma + beta
    return hprev + jnp.maximum(zn, 0.0)


def _layer_body(aggp_ref, bc_ref, a_ref, w_ref, bias_ref, g_ref, be_ref, hp_ref,
                h_ref, hs_ref):
    h = _bn_block(aggp_ref[...], bc_ref[...], w_ref[...], bias_ref[...],
                  g_ref[...], be_ref[...], hp_ref[...])
    h_ref[...] = h
    hs_ref[...] = h * a_ref[...]


def _layer_tc(aggp, bcol, acol, w, bias, gamma, beta, hprev):
    return pl.pallas_call(
        _layer_body,
        out_shape=(
            jax.ShapeDtypeStruct((_N, _D), _f32),
            jax.ShapeDtypeStruct((_N, _D), _f32),
        ),
    )(aggp, bcol, acol, w, bias, gamma, beta, hprev)


def _final_body(aggp_ref, bc_ref, w_ref, bias_ref, g_ref, be_ref, hp_ref,
                w1_ref, b1_ref, w2_ref, b2_ref, w3_ref, b3_ref, out_ref):
    h = _bn_block(aggp_ref[...], bc_ref[...], w_ref[...], bias_ref[...],
                  g_ref[...], be_ref[...], hp_ref[...])
    r = jnp.maximum(jnp.dot(h, w1_ref[...], preferred_element_type=_f32) + b1_ref[...], 0.0)
    r = jnp.maximum(jnp.dot(r, w2_ref[...], preferred_element_type=_f32) + b2_ref[...], 0.0)
    out_ref[...] = jnp.dot(r, w3_ref[...], preferred_element_type=_f32) + b3_ref[...]


def _final_tc(aggp, bcol, w, bias, gamma, beta, hprev, mlp):
    args = [aggp, bcol, w, bias, gamma, beta, hprev]
    for lp in mlp:
        args.append(lp['W'])
        args.append(lp['b'].reshape(1, -1))
    return pl.pallas_call(
        _final_body,
        out_shape=jax.ShapeDtypeStruct((_N, 7), _f32),
    )(*args)


def kernel(feature, params, edge_index):
    zeros_nd = jnp.zeros((_NP, _D), _f32)
    mask = jnp.concatenate(
        [jnp.tile(jnp.array([[1.0, 0.0]], _f32), (_NW, 1)),
         jnp.tile(jnp.array([[0.0, 1.0]], _f32), (_NW, 1))], axis=0)

    edge_flat = edge_index.reshape(-1)
    pad_t = _EPT2 - _EPT
    pad_n = _NW * pad_t
    pad_dst = (_N + (jnp.arange(pad_n, dtype=jnp.int32) % (_NP - _N))).reshape(_NW, pad_t)
    src_pad = jnp.concatenate(
        [edge_index[0].reshape(_NW, _EPT), jnp.zeros((_NW, pad_t), jnp.int32)], axis=1)
    dst_pad = jnp.concatenate(
        [edge_index[1].reshape(_NW, _EPT), pad_dst], axis=1)
    edge_pad = jnp.concatenate([src_pad.reshape(-1), dst_pad.reshape(-1)])
    degp = _deg_sc(edge_flat).reshape(2 * _NW, _NP)

    emb = params['emb']
    h, hs, acol, bcol = _embed_tc(feature, emb['W'], emb['b'].reshape(1, _D), degp, mask)

    layers = params['layers']
    out = None
    for i, lp in enumerate(layers):
        aggp = _edge_sc(hs, edge_pad, zeros_nd).reshape(_NC, _NP, _D)
        w = lp['W']
        bias = lp['b'].reshape(1, -1)
        gamma = lp['gamma'].reshape(1, -1)
        beta = lp['beta'].reshape(1, -1)
        if i < len(layers) - 1:
            h, hs = _layer_tc(aggp, bcol, acol, w, bias, gamma, beta, h)
        else:
            out = _final_tc(aggp, bcol, w, bias, gamma, beta, h, params['mlp'])
    return out


# speedup vs baseline: 3.7971x; 3.2573x over previous
import functools

import jax
import jax.numpy as jnp
from jax import lax
from jax.experimental import pallas as pl
from jax.experimental.pallas import tpu as pltpu
from jax.experimental.pallas import tpu_sc as plsc

_N = 10000
_E = 320000
_D = 128
_NC = 2
_NS = 16
_NW = _NC * _NS
_EPT = _E // _NW
_C = 80
_NCHUNK = _EPT // _C
_C2 = 128
_NCHUNK2 = 81
_EPT2 = _C2 * _NCHUNK2
_PE = _NW * _EPT2
_NH = _N + 16
_NP = 10240
_RPT = _NP // _NS

_f32 = jnp.float32
_mesh = plsc.VectorSubcoreMesh(core_axis_name="c", subcore_axis_name="s")


_CE = 2000


@functools.partial(
    pl.kernel,
    out_type=jax.ShapeDtypeStruct((2 * _NW * _NP,), _f32),
    mesh=_mesh,
    scratch_types=[
        pltpu.VMEM((2 * _NP,), _f32),
        pltpu.VMEM((_CE,), jnp.int32),
        pltpu.VMEM((_CE,), jnp.int32),
    ],
    compiler_params=pltpu.CompilerParams(needs_layout_passes=False),
)
def _deg_sc(edge_hbm, out_hbm, deg_v, sidx, didx):
    c = lax.axis_index("c")
    s = lax.axis_index("s")
    tid = c * _NS + s
    zero16 = jnp.zeros((16,), _f32)
    one16 = jnp.full((16,), 1.0, _f32)
    offN = jnp.full((16,), _NP, jnp.int32)

    def zb(i, carry):
        deg_v[pl.ds(i * 16, 16)] = zero16
        return carry

    lax.fori_loop(0, 2 * _NP // 16, zb, 0)

    def step(k, carry):
        base = tid * _EPT + k * _CE
        pltpu.sync_copy(edge_hbm.at[pl.ds(base, _CE)], sidx)
        pltpu.sync_copy(edge_hbm.at[pl.ds(_E + base, _CE)], didx)

        def inner(j, c2):
            plsc.addupdate_scatter(deg_v, [sidx[pl.ds(j * 16, 16)]], one16)
            plsc.addupdate_scatter(deg_v, [didx[pl.ds(j * 16, 16)] + offN], one16)
            return c2

        lax.fori_loop(0, _CE // 16, inner, 0)
        return carry

    lax.fori_loop(0, _EPT // _CE, step, 0)
    pltpu.sync_copy(deg_v.at[pl.ds(0, _NP)], out_hbm.at[pl.ds(tid * _NP, _NP)])
    pltpu.sync_copy(deg_v.at[pl.ds(_NP, _NP)], out_hbm.at[pl.ds((_NW + tid) * _NP, _NP)])


@functools.partial(
    pl.kernel,
    out_type=jax.ShapeDtypeStruct((_NC * _NP, _D), _f32),
    mesh=_mesh,
    scratch_types=[
        pltpu.VMEM_SHARED((_NP, _D), _f32),
        pltpu.VMEM((_NCHUNK2, _C2), jnp.int32),
        pltpu.VMEM((_C2,), jnp.int32),
        pltpu.VMEM((_C2,), jnp.int32),
        pltpu.VMEM((_C2, _D), _f32),
        pltpu.VMEM((_C2, _D), _f32),
        pltpu.SemaphoreType.DMA,
        pltpu.SemaphoreType.DMA,
        pltpu.SemaphoreType.DMA,
        pltpu.SemaphoreType.DMA,
        pltpu.SemaphoreType.DMA,
        pltpu.SemaphoreType.DMA,
        pltpu.SemaphoreType.DMA,
    ],
)
def _edge_sc(hs_hbm, edge_hbm, zeros_hbm, out_hbm, acc, didx, sb0, sb1,
             rows0, rows1, isem, xsem0, xsem1, gsem0, gsem1, ssem0, ssem1):
    c = lax.axis_index("c")
    s = lax.axis_index("s")
    tid = c * _NS + s
    ebase = tid * _EPT2

    def fire_didx(k, carry):
        pltpu.async_copy(edge_hbm.at[pl.ds(_PE + ebase + k * _C2, _C2)], didx.at[k], isem)
        return carry

    lax.fori_loop(0, _NCHUNK2, fire_didx, 0)
    pltpu.sync_copy(zeros_hbm.at[pl.ds(s * _RPT, _RPT)], acc.at[pl.ds(s * _RPT, _RPT)])

    def drain_didx(k, carry):
        pltpu.make_async_copy(edge_hbm.at[pl.ds(_PE + ebase + k * _C2, _C2)],
                              didx.at[k], isem).wait()
        return carry

    lax.fori_loop(0, _NCHUNK2, drain_didx, 0)
    plsc.subcore_barrier()

    def load_sidx(k, buf, sem):
        return pltpu.async_copy(edge_hbm.at[pl.ds(ebase + k * _C2, _C2)], buf, sem)

    def drain_sidx(buf, sem):
        pltpu.make_async_copy(edge_hbm.at[pl.ds(0, _C2)], buf, sem).wait()

    def gather(sbuf, buf, sem):
        return pltpu.async_copy(hs_hbm.at[sbuf], buf, sem)

    def scatter(k, buf, sem):
        return pltpu.async_copy(buf, acc.at[didx.at[k]], sem, add=True)

    def drain(buf, sem):
        pltpu.make_async_copy(hs_hbm.at[pl.ds(0, _C2)], buf, sem).wait()

    load_sidx(0, sb0, xsem0).wait()
    load_sidx(1, sb1, xsem1)
    gather(sb0, rows0, gsem0)

    def body(t, carry):
        k0 = 2 * t
        drain(rows0, gsem0)
        load_sidx(k0 + 2, sb0, xsem0)
        s0 = scatter(k0, rows0, ssem0)

        @pl.when(t > 0)
        def _():
            drain(rows1, ssem1)

        drain_sidx(sb1, xsem1)
        g1 = gather(sb1, rows1, gsem1)
        g1.wait()
        load_sidx(k0 + 3, sb1, xsem1)
        scatter(k0 + 1, rows1, ssem1)
        s0.wait()
        drain_sidx(sb0, xsem0)
        gather(sb0, rows0, gsem0)
        return carry

    lax.fori_loop(0, (_NCHUNK2 - 1) // 2, body, 0)

    kl = _NCHUNK2 - 1
    drain(rows0, gsem0)
    sl = scatter(kl, rows0, ssem0)
    drain(rows1, ssem1)
    drain_sidx(sb1, xsem1)
    sl.wait()
    plsc.subcore_barrier()
    pltpu.sync_copy(acc.at[pl.ds(s * _RPT, _RPT)],
                    out_hbm.at[pl.ds(c * _NP + s * _RPT, _RPT)])


def _embed_body(f_ref, w_ref, b_ref, degp_ref, m_ref, h_ref, hs_ref, a_ref, bc_ref):
    h = jnp.dot(f_ref[...], w_ref[...], preferred_element_type=_f32) + b_ref[...]
    d = lax.dot_general(degp_ref[...], m_ref[...],
                        ((( 0,), (0,)), ((), ())),
                        preferred_element_type=_f32)
    a = lax.rsqrt(jnp.maximum(d[:_N, 0:1], 1.0))
    b = lax.rsqrt(jnp.maximum(d[:_N, 1:2], 1.0))
    h_ref[...] = h
    hs_ref[0:_N, :] = h * a
    hs_ref[_N:_NH, :] = jnp.zeros((_NH - _N, _D), _f32)
    a_ref[...] = a
    bc_ref[...] = b


def _embed_tc(feature, w, b2d, degp, mask):
    return pl.pallas_call(
        _embed_body,
        out_shape=(
            jax.ShapeDtypeStruct((_N, _D), _f32),
            jax.ShapeDtypeStruct((_NH, _D), _f32),
            jax.ShapeDtypeStruct((_N, 1), _f32),
            jax.ShapeDtypeStruct((_N, 1), _f32),
        ),
    )(feature, w, b2d, degp, mask)


def _bn_block(aggp, bcol, w, bias, gamma, beta, hprev):
    agg = (aggp[0, :_N] + aggp[1, :_N]) * bcol
    z = jnp.dot(agg, w, preferred_element_type=_f32) + bias
    mu = jnp.mean(z, axis=0, keepdims=True)
    zc = z - mu
    var = jnp.mean(zc * zc, axis=0, keepdims=True)
    zn = zc * lax.rsqrt(var + 1e-5) * gamma + beta
    return hprev + jnp.maximum(zn, 0.0)


def _layer_body(aggp_ref, bc_ref, a_ref, w_ref, bias_ref, g_ref, be_ref, hp_ref,
                h_ref, hs_ref):
    h = _bn_block(aggp_ref[...], bc_ref[...], w_ref[...], bias_ref[...],
                  g_ref[...], be_ref[...], hp_ref[...])
    h_ref[...] = h
    hs_ref[0:_N, :] = h * a_ref[...]
    hs_ref[_N:_NH, :] = jnp.zeros((_NH - _N, _D), _f32)


def _layer_tc(aggp, bcol, acol, w, bias, gamma, beta, hprev):
    return pl.pallas_call(
        _layer_body,
        out_shape=(
            jax.ShapeDtypeStruct((_N, _D), _f32),
            jax.ShapeDtypeStruct((_NH, _D), _f32),
        ),
    )(aggp, bcol, acol, w, bias, gamma, beta, hprev)


def _final_body(aggp_ref, bc_ref, w_ref, bias_ref, g_ref, be_ref, hp_ref,
                w1_ref, b1_ref, w2_ref, b2_ref, w3_ref, b3_ref, out_ref):
    h = _bn_block(aggp_ref[...], bc_ref[...], w_ref[...], bias_ref[...],
                  g_ref[...], be_ref[...], hp_ref[...])
    r = jnp.maximum(jnp.dot(h, w1_ref[...], preferred_element_type=_f32) + b1_ref[...], 0.0)
    r = jnp.maximum(jnp.dot(r, w2_ref[...], preferred_element_type=_f32) + b2_ref[...], 0.0)
    out_ref[...] = jnp.dot(r, w3_ref[...], preferred_element_type=_f32) + b3_ref[...]


def _final_tc(aggp, bcol, w, bias, gamma, beta, hprev, mlp):
    args = [aggp, bcol, w, bias, gamma, beta, hprev]
    for lp in mlp:
        args.append(lp['W'])
        args.append(lp['b'].reshape(1, -1))
    return pl.pallas_call(
        _final_body,
        out_shape=jax.ShapeDtypeStruct((_N, 7), _f32),
    )(*args)


def kernel(feature, params, edge_index):
    zeros_nd = jnp.zeros((_NP, _D), _f32)
    mask = jnp.concatenate(
        [jnp.tile(jnp.array([[1.0, 0.0]], _f32), (_NW, 1)),
         jnp.tile(jnp.array([[0.0, 1.0]], _f32), (_NW, 1))], axis=0)

    edge_flat = edge_index.reshape(-1)
    pad_t = _EPT2 - _EPT
    pad_n = _NW * pad_t
    pad_src = (_N + (jnp.arange(pad_n, dtype=jnp.int32) % (_NH - _N))).reshape(_NW, pad_t)
    pad_dst = (jnp.arange(pad_n, dtype=jnp.int32) * 97 % _NP).reshape(_NW, pad_t)
    src_pad = jnp.concatenate([edge_index[0].reshape(_NW, _EPT), pad_src], axis=1)
    dst_pad = jnp.concatenate([edge_index[1].reshape(_NW, _EPT), pad_dst], axis=1)
    edge_pad = jnp.concatenate([src_pad.reshape(-1), dst_pad.reshape(-1)])
    degp = _deg_sc(edge_flat).reshape(2 * _NW, _NP)

    emb = params['emb']
    h, hs, acol, bcol = _embed_tc(feature, emb['W'], emb['b'].reshape(1, _D), degp, mask)

    layers = params['layers']
    out = None
    for i, lp in enumerate(layers):
        aggp = _edge_sc(hs, edge_pad, zeros_nd).reshape(_NC, _NP, _D)
        w = lp['W']
        bias = lp['b'].reshape(1, -1)
        gamma = lp['gamma'].reshape(1, -1)
        beta = lp['beta'].reshape(1, -1)
        if i < len(layers) - 1:
            h, hs = _layer_tc(aggp, bcol, acol, w, bias, gamma, beta, h)
        else:
            out = _final_tc(aggp, bcol, w, bias, gamma, beta, h, params['mlp'])
    return out


# speedup vs baseline: 4.1863x; 1.1025x over previous
import functools

import jax
import jax.numpy as jnp
from jax import lax
from jax.experimental import pallas as pl
from jax.experimental.pallas import tpu as pltpu
from jax.experimental.pallas import tpu_sc as plsc

_N = 10000
_E = 320000
_D = 128
_NC = 2
_NS = 16
_NW = _NC * _NS
_EPT = _E // _NW
_C = 80
_NCHUNK = _EPT // _C
_C2 = 120
_NCHUNK2 = 87
_EPT2 = _C2 * _NCHUNK2
_PE = _NW * _EPT2
_NH = _N + 16
_NP = 10112
_RPT = _NP // _NS

_f32 = jnp.float32
_mesh = plsc.VectorSubcoreMesh(core_axis_name="c", subcore_axis_name="s")


_CE = 2000


@functools.partial(
    pl.kernel,
    out_type=jax.ShapeDtypeStruct((2 * _NW * _NP,), _f32),
    mesh=_mesh,
    scratch_types=[
        pltpu.VMEM((2 * _NP,), _f32),
        pltpu.VMEM((_CE,), jnp.int32),
        pltpu.VMEM((_CE,), jnp.int32),
    ],
    compiler_params=pltpu.CompilerParams(needs_layout_passes=False),
)
def _deg_sc(edge_hbm, out_hbm, deg_v, sidx, didx):
    c = lax.axis_index("c")
    s = lax.axis_index("s")
    tid = c * _NS + s
    zero16 = jnp.zeros((16,), _f32)
    one16 = jnp.full((16,), 1.0, _f32)
    offN = jnp.full((16,), _NP, jnp.int32)

    def zb(i, carry):
        deg_v[pl.ds(i * 16, 16)] = zero16
        return carry

    lax.fori_loop(0, 2 * _NP // 16, zb, 0)

    def step(k, carry):
        base = tid * _EPT + k * _CE
        pltpu.sync_copy(edge_hbm.at[pl.ds(base, _CE)], sidx)
        pltpu.sync_copy(edge_hbm.at[pl.ds(_E + base, _CE)], didx)

        def inner(j, c2):
            plsc.addupdate_scatter(deg_v, [sidx[pl.ds(j * 16, 16)]], one16)
            plsc.addupdate_scatter(deg_v, [didx[pl.ds(j * 16, 16)] + offN], one16)
            return c2

        lax.fori_loop(0, _CE // 16, inner, 0)
        return carry

    lax.fori_loop(0, _EPT // _CE, step, 0)
    pltpu.sync_copy(deg_v.at[pl.ds(0, _NP)], out_hbm.at[pl.ds(tid * _NP, _NP)])
    pltpu.sync_copy(deg_v.at[pl.ds(_NP, _NP)], out_hbm.at[pl.ds((_NW + tid) * _NP, _NP)])


@functools.partial(
    pl.kernel,
    out_type=jax.ShapeDtypeStruct((_NC * _NP, _D), _f32),
    mesh=_mesh,
    scratch_types=[
        pltpu.VMEM_SHARED((_NP, _D), _f32),
        pltpu.VMEM((_C2,), jnp.int32),
        pltpu.VMEM((_C2,), jnp.int32),
        pltpu.VMEM((_C2,), jnp.int32),
        pltpu.VMEM((_C2,), jnp.int32),
        pltpu.VMEM((_C2,), jnp.int32),
        pltpu.VMEM((_C2,), jnp.int32),
        pltpu.VMEM((_C2, _D), _f32),
        pltpu.VMEM((_C2, _D), _f32),
        pltpu.VMEM((_C2, _D), _f32),
        pltpu.SemaphoreType.DMA,
        pltpu.SemaphoreType.DMA,
        pltpu.SemaphoreType.DMA,
        pltpu.SemaphoreType.DMA,
        pltpu.SemaphoreType.DMA,
        pltpu.SemaphoreType.DMA,
        pltpu.SemaphoreType.DMA,
        pltpu.SemaphoreType.DMA,
        pltpu.SemaphoreType.DMA,
        pltpu.SemaphoreType.DMA,
        pltpu.SemaphoreType.DMA,
        pltpu.SemaphoreType.DMA,
    ],
)
def _edge_sc(hs_hbm, edge_hbm, zeros_hbm, out_hbm, acc,
             sb0, sb1, sb2, db0, db1, db2, rows0, rows1, rows2,
             xsem0, xsem1, xsem2, dsem0, dsem1, dsem2,
             gsem0, gsem1, gsem2, ssem0, ssem1, ssem2):
    c = lax.axis_index("c")
    s = lax.axis_index("s")
    tid = c * _NS + s
    ebase = tid * _EPT2

    pltpu.sync_copy(zeros_hbm.at[pl.ds(s * _RPT, _RPT)], acc.at[pl.ds(s * _RPT, _RPT)])
    plsc.subcore_barrier()

    def load_sidx(k, buf, sem):
        return pltpu.async_copy(edge_hbm.at[pl.ds(ebase + k * _C2, _C2)], buf, sem)

    def load_didx(k, buf, sem):
        return pltpu.async_copy(edge_hbm.at[pl.ds(_PE + ebase + k * _C2, _C2)], buf, sem)

    def drain_idx(buf, sem):
        pltpu.make_async_copy(edge_hbm.at[pl.ds(0, _C2)], buf, sem).wait()

    def gather(sbuf, buf, sem):
        return pltpu.async_copy(hs_hbm.at[sbuf], buf, sem)

    def scatter(dbuf, buf, sem):
        return pltpu.async_copy(buf, acc.at[dbuf], sem, add=True)

    def drain(buf, sem):
        pltpu.make_async_copy(hs_hbm.at[pl.ds(0, _C2)], buf, sem).wait()

    sb = [sb0, sb1, sb2]
    db = [db0, db1, db2]
    rows = [rows0, rows1, rows2]
    xsem = [xsem0, xsem1, xsem2]
    dsem = [dsem0, dsem1, dsem2]
    gsem = [gsem0, gsem1, gsem2]
    ssem = [ssem0, ssem1, ssem2]

    load_sidx(0, sb0, xsem0)
    load_sidx(1, sb1, xsem1)
    load_sidx(2, sb2, xsem2)
    load_didx(0, db0, dsem0)
    load_didx(1, db1, dsem1)
    drain_idx(sb0, xsem0)
    gather(sb0, rows0, gsem0)
    drain_idx(sb1, xsem1)
    gather(sb1, rows1, gsem1)

    def body(t, carry):
        for j in range(3):
            k = 3 * t + j
            j2 = (j + 2) % 3
            drain(rows[j], gsem[j])

            @pl.when(k + 3 < _NCHUNK2)
            def _():
                load_sidx(k + 3, sb[j], xsem[j])

            drain_idx(db[j], dsem[j])
            scatter(db[j], rows[j], ssem[j])

            @pl.when(k > 0)
            def _():
                drain(rows[j2], ssem[j2])

            @pl.when(k + 2 < _NCHUNK2)
            def _():
                load_didx(k + 2, db[j2], dsem[j2])
                drain_idx(sb[j2], xsem[j2])
                gather(sb[j2], rows[j2], gsem[j2])
        return carry

    lax.fori_loop(0, _NCHUNK2 // 3, body, 0)
    drain(rows[(_NCHUNK2 - 1) % 3], ssem[(_NCHUNK2 - 1) % 3])
    plsc.subcore_barrier()
    pltpu.sync_copy(acc.at[pl.ds(s * _RPT, _RPT)],
                    out_hbm.at[pl.ds(c * _NP + s * _RPT, _RPT)])


def _embed_body(f_ref, w_ref, b_ref, degp_ref, m_ref, h_ref, hs_ref, a_ref, bc_ref):
    h = jnp.dot(f_ref[...], w_ref[...], preferred_element_type=_f32) + b_ref[...]
    d = lax.dot_general(degp_ref[...], m_ref[...],
                        ((( 0,), (0,)), ((), ())),
                        preferred_element_type=_f32)
    a = lax.rsqrt(jnp.maximum(d[:_N, 0:1], 1.0))
    b = lax.rsqrt(jnp.maximum(d[:_N, 1:2], 1.0))
    h_ref[...] = h
    hs_ref[0:_N, :] = h * a
    hs_ref[_N:_NH, :] = jnp.zeros((_NH - _N, _D), _f32)
    a_ref[...] = a
    bc_ref[...] = b


def _embed_tc(feature, w, b2d, degp, mask):
    return pl.pallas_call(
        _embed_body,
        out_shape=(
            jax.ShapeDtypeStruct((_N, _D), _f32),
            jax.ShapeDtypeStruct((_NH, _D), _f32),
            jax.ShapeDtypeStruct((_N, 1), _f32),
            jax.ShapeDtypeStruct((_N, 1), _f32),
        ),
    )(feature, w, b2d, degp, mask)


def _bn_block(aggp, bcol, w, bias, gamma, beta, hprev):
    agg = (aggp[0, :_N] + aggp[1, :_N]) * bcol
    z = jnp.dot(agg, w, preferred_element_type=_f32) + bias
    mu = jnp.mean(z, axis=0, keepdims=True)
    zc = z - mu
    var = jnp.mean(zc * zc, axis=0, keepdims=True)
    zn = zc * lax.rsqrt(var + 1e-5) * gamma + beta
    return hprev + jnp.maximum(zn, 0.0)


def _layer_body(aggp_ref, bc_ref, a_ref, w_ref, bias_ref, g_ref, be_ref, hp_ref,
                h_ref, hs_ref):
    h = _bn_block(aggp_ref[...], bc_ref[...], w_ref[...], bias_ref[...],
                  g_ref[...], be_ref[...], hp_ref[...])
    h_ref[...] = h
    hs_ref[0:_N, :] = h * a_ref[...]
    hs_ref[_N:_NH, :] = jnp.zeros((_NH - _N, _D), _f32)


def _layer_tc(aggp, bcol, acol, w, bias, gamma, beta, hprev):
    return pl.pallas_call(
        _layer_body,
        out_shape=(
            jax.ShapeDtypeStruct((_N, _D), _f32),
            jax.ShapeDtypeStruct((_NH, _D), _f32),
        ),
    )(aggp, bcol, acol, w, bias, gamma, beta, hprev)


def _final_body(aggp_ref, bc_ref, w_ref, bias_ref, g_ref, be_ref, hp_ref,
                w1_ref, b1_ref, w2_ref, b2_ref, w3_ref, b3_ref, out_ref):
    h = _bn_block(aggp_ref[...], bc_ref[...], w_ref[...], bias_ref[...],
                  g_ref[...], be_ref[...], hp_ref[...])
    r = jnp.maximum(jnp.dot(h, w1_ref[...], preferred_element_type=_f32) + b1_ref[...], 0.0)
    r = jnp.maximum(jnp.dot(r, w2_ref[...], preferred_element_type=_f32) + b2_ref[...], 0.0)
    out_ref[...] = jnp.dot(r, w3_ref[...], preferred_element_type=_f32) + b3_ref[...]


def _final_tc(aggp, bcol, w, bias, gamma, beta, hprev, mlp):
    args = [aggp, bcol, w, bias, gamma, beta, hprev]
    for lp in mlp:
        args.append(lp['W'])
        args.append(lp['b'].reshape(1, -1))
    return pl.pallas_call(
        _final_body,
        out_shape=jax.ShapeDtypeStruct((_N, 7), _f32),
    )(*args)


def kernel(feature, params, edge_index):
    zeros_nd = jnp.zeros((_NP, _D), _f32)
    mask = jnp.concatenate(
        [jnp.tile(jnp.array([[1.0, 0.0]], _f32), (_NW, 1)),
         jnp.tile(jnp.array([[0.0, 1.0]], _f32), (_NW, 1))], axis=0)

    edge_flat = edge_index.reshape(-1)
    pad_t = _EPT2 - _EPT
    pad_n = _NW * pad_t
    pad_src = (_N + (jnp.arange(pad_n, dtype=jnp.int32) % (_NH - _N))).reshape(_NW, pad_t)
    pad_dst = (jnp.arange(pad_n, dtype=jnp.int32) * 97 % _NP).reshape(_NW, pad_t)
    src_pad = jnp.concatenate([edge_index[0].reshape(_NW, _EPT), pad_src], axis=1)
    dst_pad = jnp.concatenate([edge_index[1].reshape(_NW, _EPT), pad_dst], axis=1)
    edge_pad = jnp.concatenate([src_pad.reshape(-1), dst_pad.reshape(-1)])
    degp = _deg_sc(edge_flat).reshape(2 * _NW, _NP)

    emb = params['emb']
    h, hs, acol, bcol = _embed_tc(feature, emb['W'], emb['b'].reshape(1, _D), degp, mask)

    layers = params['layers']
    out = None
    for i, lp in enumerate(layers):
        aggp = _edge_sc(hs, edge_pad, zeros_nd).reshape(_NC, _NP, _D)
        w = lp['W']
        bias = lp['b'].reshape(1, -1)
        gamma = lp['gamma'].reshape(1, -1)
        beta = lp['beta'].reshape(1, -1)
        if i < len(layers) - 1:
            h, hs = _layer_tc(aggp, bcol, acol, w, bias, gamma, beta, h)
        else:
            out = _final_tc(aggp, bcol, w, bias, gamma, beta, h, params['mlp'])
    return out
